# Initial kernel scaffold; baseline (speedup 1.0000x reference)
import functools

import jax
import jax.numpy as jnp
from jax import lax
from jax.experimental import pallas as pl
from jax.experimental.pallas import tpu as pltpu
from jax.experimental.pallas import tpu_sc as plsc

_INFO = plsc.get_sparse_core_info()
_NC = _INFO.num_cores
_NS = _INFO.num_subcores
_NW = _NC * _NS
_LANES = _INFO.num_lanes


def kernel(user_id, item_id, hist_item, price, user_table, item_table,
           hist_table):
    B = user_id.shape[0]
    L = hist_item.shape[1]
    D = user_table.shape[1]
    OUTW = 3 * D + 1
    RPW = B // _NW
    CB = 16
    NCH = RPW // CB
    NH = D // _LANES

    mesh = plsc.VectorSubcoreMesh(core_axis_name="c", subcore_axis_name="s")

    @functools.partial(
        pl.kernel,
        out_type=jax.ShapeDtypeStruct((B, OUTW), jnp.float32),
        mesh=mesh,
        scratch_types=[
            pltpu.VMEM((RPW // 128, 128), jnp.int32),
            pltpu.VMEM((RPW // 128, 128), jnp.int32),
            pltpu.VMEM((RPW, L), jnp.int32),
            pltpu.VMEM((RPW,), jnp.float32),
            pltpu.VMEM((RPW, D), jnp.float32),
            pltpu.VMEM((RPW, D), jnp.float32),
            pltpu.VMEM((CB, L, D), jnp.float32),
            pltpu.VMEM((CB, OUTW), jnp.float32),
            pltpu.SemaphoreType.DMA,
        ],
    )
    def _emb(uid, iid, hid, pr, ut, it, ht, out,
             uidx, iidx, hidx, pst, su, si, hbuf, st, sem):
        wid = lax.axis_index("s") * _NC + lax.axis_index("c")
        base = wid * RPW

        cps = []
        for j in range(RPW // 128):
            cps.append(pltpu.async_copy(
                uid.at[pl.ds(base + j * 128, 128)], uidx.at[j], sem))
            cps.append(pltpu.async_copy(
                iid.at[pl.ds(base + j * 128, 128)], iidx.at[j], sem))
        cps.append(pltpu.async_copy(hid.at[pl.ds(base, RPW)], hidx, sem))
        cps.append(pltpu.async_copy(pr.at[pl.ds(base, RPW)], pst, sem))
        for c in cps:
            c.wait()

        gps = []
        for j in range(RPW // 128):
            gps.append(pltpu.async_copy(
                ut.at[uidx.at[j]], su.at[pl.ds(j * 128, 128)], sem))
            gps.append(pltpu.async_copy(
                it.at[iidx.at[j]], si.at[pl.ds(j * 128, 128)], sem))
        for c in gps:
            c.wait()

        iota16 = lax.broadcasted_iota(jnp.int32, (_LANES,), 0)
        col_last = jnp.full((_LANES,), OUTW - 1, jnp.int32)

        def chunk(g, _):
            r0 = g * CB
            pltpu.async_copy(ht.at[hidx.at[pl.ds(r0, CB)]], hbuf, sem).wait()

            def crow(c, _):
                r = r0 + c
                for h in range(NH):
                    a = hbuf[c, 0, pl.ds(h * _LANES, _LANES)]
                    for l in range(1, L):
                        a = a + hbuf[c, l, pl.ds(h * _LANES, _LANES)]
                    st[c, pl.ds(2 * D + h * _LANES, _LANES)] = a
                    st[c, pl.ds(h * _LANES, _LANES)] = \
                        su[r, pl.ds(h * _LANES, _LANES)]
                    st[c, pl.ds(D + h * _LANES, _LANES)] = \
                        si[r, pl.ds(h * _LANES, _LANES)]
                return 0

            lax.fori_loop(0, CB, crow, 0)
            pv = pst[pl.ds(r0, _LANES)]
            plsc.store_scatter(st, [iota16, col_last], pv)
            pltpu.sync_copy(st, out.at[pl.ds(base + r0, CB)])
            return 0

        lax.fori_loop(0, NCH, chunk, 0)

    return _emb(user_id, item_id, hist_item, price, user_table, item_table,
                hist_table)


# speedup vs baseline: 9.9546x; 9.9546x over previous
import functools

import jax
import jax.numpy as jnp
from jax import lax
from jax.experimental import pallas as pl
from jax.experimental.pallas import tpu as pltpu
from jax.experimental.pallas import tpu_sc as plsc

_INFO = plsc.get_sparse_core_info()
_NC = _INFO.num_cores
_NS = _INFO.num_subcores
_NW = _NC * _NS
_LANES = _INFO.num_lanes


def kernel(user_id, item_id, hist_item, price, user_table, item_table,
           hist_table):
    B = user_id.shape[0]
    L = hist_item.shape[1]
    D = user_table.shape[1]
    OUTW = 3 * D + 1
    RPW = B // _NW
    CB = 16
    NCH = RPW // CB
    NH = D // _LANES

    mesh = plsc.VectorSubcoreMesh(core_axis_name="c", subcore_axis_name="s")

    @functools.partial(
        pl.kernel,
        out_type=jax.ShapeDtypeStruct((B, OUTW), jnp.float32),
        mesh=mesh,
        compiler_params=pltpu.CompilerParams(
            needs_layout_passes=False, use_tc_tiling_on_sc=False),
        scratch_types=[
            pltpu.VMEM((RPW,), jnp.int32),
            pltpu.VMEM((RPW,), jnp.int32),
            pltpu.VMEM((RPW, L), jnp.int32),
            pltpu.VMEM((CB,), jnp.float32),
            pltpu.VMEM((CB, D), jnp.float32),
            pltpu.VMEM((CB, D), jnp.float32),
            pltpu.VMEM((CB, L, D), jnp.float32),
            pltpu.VMEM((CB, OUTW), jnp.float32),
            pltpu.SemaphoreType.DMA,
        ],
    )
    def _emb(uid, iid, hid, pr, ut, it, ht, out,
             uidx, iidx, hidx, pc, su, si, hbuf, st, sem):
        wid = lax.axis_index("s") * _NC + lax.axis_index("c")
        base = wid * RPW

        cps = [
            pltpu.async_copy(uid.at[pl.ds(base, RPW)], uidx, sem),
            pltpu.async_copy(iid.at[pl.ds(base, RPW)], iidx, sem),
            pltpu.async_copy(hid.at[pl.ds(base, RPW)], hidx, sem),
        ]
        for c in cps:
            c.wait()

        iota16 = lax.broadcasted_iota(jnp.int32, (_LANES,), 0)
        col_last = jnp.full((_LANES,), OUTW - 1, jnp.int32)

        def chunk(g, _):
            r0 = g * CB
            dps = [pltpu.async_copy(ht.at[hidx.at[r0 + c]], hbuf.at[c], sem)
                   for c in range(CB)]
            dps.append(pltpu.async_copy(
                ut.at[uidx.at[pl.ds(r0, CB)]], su, sem))
            dps.append(pltpu.async_copy(
                it.at[iidx.at[pl.ds(r0, CB)]], si, sem))
            dps.append(pltpu.async_copy(pr.at[pl.ds(base + r0, CB)], pc, sem))
            for c in dps:
                c.wait()

            for c in range(CB):
                for h in range(NH):
                    o = h * _LANES
                    a = hbuf[c, 0, pl.ds(o, _LANES)]
                    for l in range(1, L):
                        a = a + hbuf[c, l, pl.ds(o, _LANES)]
                    st[c, pl.ds(2 * D + o, _LANES)] = a
                    st[c, pl.ds(o, _LANES)] = su[c, pl.ds(o, _LANES)]
                    st[c, pl.ds(D + o, _LANES)] = si[c, pl.ds(o, _LANES)]

            plsc.store_scatter(st, [iota16, col_last], pc[...])
            pltpu.sync_copy(st, out.at[pl.ds(base + r0, CB)])
            return 0

        lax.fori_loop(0, NCH, chunk, 0)

    return _emb(user_id, item_id, hist_item, price, user_table, item_table,
                hist_table)
